# Initial kernel scaffold; baseline (speedup 1.0000x reference)
import jax
import jax.numpy as jnp
from jax.experimental import pallas as pl


def kernel(x, edge_index, W1, a_src1, a_dst1, b1, W2, a_src2, a_dst2, b2):
    raise NotImplementedError("write your pallas kernel here")


# speedup vs baseline: 9.1856x; 9.1856x over previous
import functools

import jax
import jax.numpy as jnp
from jax import lax
from jax.experimental import pallas as pl
from jax.experimental.pallas import tpu as pltpu
from jax.experimental.pallas import tpu_sc as plsc

N = 10000
NP = 10240
NB = NP // 256
NFEAT = 256
HEADS = 4
HID = 256
NCH = 8
CW = 128
NCLS = 64
E = 160000
E2 = E + N
EB = 128
EPT = 5376
EP = EPT * 32
NBAT = EPT // EB
NSC = 2
NSUB = 16
ROWS_PT = NP // NSUB

_f32 = jnp.float32
_i32 = jnp.int32


def _tc1_body(x_ref, w1_ref, bb_ref, h_ref, asad_ref):
    xb = x_ref[...]
    h_ref[0] = jnp.dot(xb, w1_ref[...], preferred_element_type=_f32, precision=lax.Precision.HIGHEST)

    @pl.when(pl.program_id(1) == 0)
    def _():
        asad_ref[...] = jnp.dot(xb, bb_ref[...], preferred_element_type=_f32, precision=lax.Precision.HIGHEST)[:, :8]


def _tc1(xp, w1, bbp):
    return pl.pallas_call(
        _tc1_body,
        grid=(NB, NCH),
        in_specs=[
            pl.BlockSpec((256, NFEAT), lambda i, c: (i, 0)),
            pl.BlockSpec((NFEAT, CW), lambda i, c: (0, c)),
            pl.BlockSpec((NFEAT, 128), lambda i, c: (0, 0)),
        ],
        out_specs=[
            pl.BlockSpec((1, 256, CW), lambda i, c: (c, i, 0)),
            pl.BlockSpec((256, 8), lambda i, c: (i, 0)),
        ],
        out_shape=[
            jax.ShapeDtypeStruct((NCH, NP, CW), _f32),
            jax.ShapeDtypeStruct((NP, 8), _f32),
        ],
    )(xp, w1, bbp)


def _sca_body(src_hbm, dst_hbm, asad_hbm, w_hbm, denp_hbm,
              asad_v, src_v, dst_v, wb_v, w16_v, zb_v, dacc_sh, sem):
    c = lax.axis_index("c")
    s = lax.axis_index("s")
    wid = c * NSUB + s

    pltpu.sync_copy(asad_hbm, asad_v)

    def _z1(i, _):
        zb_v[i] = jnp.zeros((16,), _f32)
        return 0
    lax.fori_loop(0, ROWS_PT, _z1, 0)

    def _z2(i, _):
        w16_v[i] = jnp.zeros((16,), _f32)
        return 0
    lax.fori_loop(0, EB, _z2, 0)

    pltpu.sync_copy(zb_v, dacc_sh.at[pl.ds(s * ROWS_PT, ROWS_PT)])
    plsc.subcore_barrier()

    def _mx(i, m):
        return jnp.maximum(m, asad_v[i])
    macc = lax.fori_loop(0, NP * 8 // 16, _mx, jnp.full((16,), -3.4e38, _f32))
    iota16 = lax.iota(_i32, 16)
    gs = []
    for h in range(HEADS):
        vh = jnp.where((iota16 == h) | (iota16 == h + 8), macc, -3.4e38)
        gs.append(plsc.cummax(lax.rev(plsc.cummax(vh), (0,))))

    def _batch(b, _):
        off = wid * EPT + b * EB
        pltpu.sync_copy(src_hbm.at[pl.ds(off, EB)], src_v)
        pltpu.sync_copy(dst_hbm.at[pl.ds(off, EB)], dst_v)
        for h in range(HEADS):
            def _grp(g, _h=h):
                s16 = src_v[pl.ds(g * 16, 16)]
                d16 = dst_v[pl.ds(g * 16, 16)]
                fs = s16 * 8 + _h
                fd = d16 * 8 + (4 + _h)
                asg = plsc.load_gather(asad_v, [fs >> 4, fs & 15])
                adg = plsc.load_gather(asad_v, [fd >> 4, fd & 15])
                al = asg + adg
                al = jnp.where(al > 0, al, 0.2 * al)
                mm = gs[_h] + adg
                mm = jnp.where(mm > 0, mm, 0.2 * mm)
                w = jnp.exp(al - mm)
                wb_v[_h, pl.ds(g * 16, 16)] = w
                plsc.store_scatter(
                    w16_v,
                    [g * 16 + iota16, jnp.full((16,), _h, _i32)],
                    w)

            def _grp_loop(g, _2, _f=_grp):
                _f(g)
                return 0
            lax.fori_loop(0, EB // 16, _grp_loop, 0)
        pltpu.sync_copy(w16_v, dacc_sh.at[dst_v], add=True)
        for h in range(HEADS):
            pltpu.sync_copy(wb_v.at[h], w_hbm.at[h, pl.ds(off, EB)])
        return 0

    lax.fori_loop(0, NBAT, _batch, 0)
    plsc.subcore_barrier()
    for j in range(ROWS_PT // EB):
        pltpu.sync_copy(dacc_sh.at[pl.ds(s * ROWS_PT + j * EB, EB)], w16_v)
        pltpu.sync_copy(
            w16_v, denp_hbm.at[pl.ds(c * NP + s * ROWS_PT + j * EB, EB)])


def _sca(src, dst, asad2d):
    mesh = plsc.VectorSubcoreMesh(core_axis_name="c", subcore_axis_name="s")
    f = pl.kernel(
        _sca_body,
        out_type=[
            jax.ShapeDtypeStruct((HEADS, EP), _f32),
            jax.ShapeDtypeStruct((NSC * NP, 16), _f32),
        ],
        mesh=mesh,
        compiler_params=pltpu.CompilerParams(needs_layout_passes=False, use_tc_tiling_on_sc=False),
        scratch_types=[
            pltpu.VMEM((NP * 8 // 16, 16), _f32),
            pltpu.VMEM((EB,), _i32),
            pltpu.VMEM((EB,), _i32),
            pltpu.VMEM((HEADS, EB), _f32),
            pltpu.VMEM((EB, 16), _f32),
            pltpu.VMEM((ROWS_PT, 16), _f32),
            pltpu.VMEM_SHARED((NP, 16), _f32),
            pltpu.SemaphoreType.DMA,
        ],
    )
    return f(src, dst, asad2d)


def _scb_body(src_hbm, dst_hbm, w_hbm, hflat_hbm, o1p_hbm,
              src_v, dst_v, idx_v, w_v, rows_v, zb_v, acc_sh, sem):
    c = lax.axis_index("c")
    s = lax.axis_index("s")

    def _z(i, _):
        for k2 in range(8):
            zb_v[i, pl.ds(k2 * 16, 16)] = jnp.zeros((16,), _f32)
        return 0
    lax.fori_loop(0, EB, _z, 0)

    def _chunk_body(chunk, _0):
        hc = chunk // 2
        for j in range(ROWS_PT // EB):
            pltpu.sync_copy(zb_v, acc_sh.at[pl.ds(s * ROWS_PT + j * EB, EB)])
        plsc.subcore_barrier()

        def _batch(b, _):
            off = c * (EP // 2) + s * EPT + b * EB
            pltpu.sync_copy(src_hbm.at[pl.ds(off, EB)], src_v)
            pltpu.sync_copy(dst_hbm.at[pl.ds(off, EB)], dst_v)
            pltpu.sync_copy(w_hbm.at[hc, pl.ds(off, EB)], w_v)

            def _ix(g, _2):
                src16 = src_v[pl.ds(g * 16, 16)]
                idx_v[pl.ds(g * 16, 16)] = src16 + chunk * NP
                return 0
            lax.fori_loop(0, EB // 16, _ix, 0)

            pltpu.async_copy(hflat_hbm.at[idx_v], rows_v, sem).wait()

            def _row(g, _2):
                wvec = w_v[pl.ds(g * 16, 16)]
                for j in range(16):
                    ws = wvec[j]
                    for k2 in range(8):
                        sl = pl.ds(k2 * 16, 16)
                        rows_v[g * 16 + j, sl] = rows_v[g * 16 + j, sl] * ws
                return 0
            lax.fori_loop(0, EB // 16, _row, 0)

            pltpu.sync_copy(rows_v, acc_sh.at[dst_v], add=True)
            return 0

        lax.fori_loop(0, NBAT, _batch, 0)
        plsc.subcore_barrier()
        base = (c * NCH + chunk) * NP + s * ROWS_PT
        for j in range(ROWS_PT // EB):
            pltpu.sync_copy(acc_sh.at[pl.ds(s * ROWS_PT + j * EB, EB)], rows_v)
            pltpu.sync_copy(rows_v, o1p_hbm.at[pl.ds(base + j * EB, EB)])
        plsc.subcore_barrier()
        return 0

    lax.fori_loop(0, NCH, _chunk_body, 0)


def _scb(src, dst, w, hflat):
    mesh = plsc.VectorSubcoreMesh(core_axis_name="c", subcore_axis_name="s")
    f = pl.kernel(
        _scb_body,
        out_type=jax.ShapeDtypeStruct((NSC * NCH * NP, CW), _f32),
        mesh=mesh,
        compiler_params=pltpu.CompilerParams(needs_layout_passes=False, use_tc_tiling_on_sc=False),
        scratch_types=[
            pltpu.VMEM((EB,), _i32),
            pltpu.VMEM((EB,), _i32),
            pltpu.VMEM((EB,), _i32),
            pltpu.VMEM((EB,), _f32),
            pltpu.VMEM((EB, CW), _f32),
            pltpu.VMEM((EB, CW), _f32),
            pltpu.VMEM_SHARED((NP, CW), _f32),
            pltpu.SemaphoreType.DMA,
        ],
    )
    return f(src, dst, w, hflat)


def _tc2_body(o1a_ref, o1b_ref, dpa_ref, dpb_ref, b1_ref, w2_ref,
              h2_ref, asad2_ref):
    k = pl.program_id(1)
    o1 = o1a_ref[...] + o1b_ref[...]
    dall = dpa_ref[...] + dpb_ref[...]
    hk = k // 2
    lane16 = lax.broadcasted_iota(_i32, (256, 16), 1)
    d = jnp.sum(jnp.where(lane16 == hk, dall, 0.0), axis=1, keepdims=True)
    hb = o1 / (d + 1e-16) + b1_ref[0]
    hb = jnp.where(hb > 0, hb, jnp.exp(jnp.minimum(hb, 0.0)) - 1.0)
    mm = jnp.dot(hb, w2_ref[...], preferred_element_type=_f32, precision=lax.Precision.HIGHEST)

    @pl.when(k == 0)
    def _():
        h2_ref[...] = mm

    @pl.when(k > 0)
    def _():
        h2_ref[...] += mm

    @pl.when(k == NCH - 1)
    def _():
        asad2_ref[...] = h2_ref[:, 64:66]


def _tc2(o1p, denp, b1r, w2cat):
    return pl.pallas_call(
        _tc2_body,
        grid=(NB, NCH),
        in_specs=[
            pl.BlockSpec((256, CW), lambda i, k: (k * NB + i, 0)),
            pl.BlockSpec((256, CW), lambda i, k: ((NCH + k) * NB + i, 0)),
            pl.BlockSpec((256, 16), lambda i, k: (i, 0)),
            pl.BlockSpec((256, 16), lambda i, k: (NB + i, 0)),
            pl.BlockSpec((1, 1, 128), lambda i, k: (k, 0, 0)),
            pl.BlockSpec((CW, 128), lambda i, k: (k, 0)),
        ],
        out_specs=[
            pl.BlockSpec((256, 128), lambda i, k: (i, 0)),
            pl.BlockSpec((256, 2), lambda i, k: (i, 0)),
        ],
        out_shape=[
            jax.ShapeDtypeStruct((NP, 128), _f32),
            jax.ShapeDtypeStruct((NP, 2), _f32),
        ],
    )(o1p, o1p, denp, denp, b1r, w2cat)


def _scc_body(src_hbm, dst_hbm, asad_hbm, h2_hbm, o2p_hbm,
              asad_v, src_v, dst_v, w_v, rows_v, acc_sh, sem):
    c = lax.axis_index("c")
    s = lax.axis_index("s")

    pltpu.sync_copy(asad_hbm, asad_v)

    def _z(i, _):
        for k2 in range(8):
            rows_v[i, pl.ds(k2 * 16, 16)] = jnp.zeros((16,), _f32)
        return 0
    lax.fori_loop(0, EB, _z, 0)
    for j in range(ROWS_PT // EB):
        pltpu.sync_copy(rows_v, acc_sh.at[pl.ds(s * ROWS_PT + j * EB, EB)])
    plsc.subcore_barrier()

    def _mx(i, m):
        return jnp.maximum(m, asad_v[i])
    macc = lax.fori_loop(0, NP * 2 // 16, _mx, jnp.full((16,), -3.4e38, _f32))
    iota16 = lax.iota(_i32, 16)
    v2 = jnp.where(iota16 % 2 == 0, macc, -3.4e38)
    g2 = plsc.cummax(lax.rev(plsc.cummax(v2), (0,)))

    def _batch(b, _):
        off = c * (EP // 2) + s * EPT + b * EB
        pltpu.sync_copy(src_hbm.at[pl.ds(off, EB)], src_v)
        pltpu.sync_copy(dst_hbm.at[pl.ds(off, EB)], dst_v)
        pltpu.async_copy(h2_hbm.at[src_v], rows_v, sem).wait()

        def _grp(g, _2):
            s16 = src_v[pl.ds(g * 16, 16)]
            d16 = dst_v[pl.ds(g * 16, 16)]
            fs = s16 * 2
            fd = d16 * 2 + 1
            asg = plsc.load_gather(asad_v, [fs >> 4, fs & 15])
            adg = plsc.load_gather(asad_v, [fd >> 4, fd & 15])
            al = asg + adg
            al = jnp.where(al > 0, al, 0.2 * al)
            mm = g2 + adg
            mm = jnp.where(mm > 0, mm, 0.2 * mm)
            w = jnp.exp(al - mm)
            w_v[pl.ds(g * 16, 16)] = w
            plsc.store_scatter(
                rows_v, [g * 16 + iota16, jnp.full((16,), 64, _i32)], w)
            return 0
        lax.fori_loop(0, EB // 16, _grp, 0)

        def _row(g, _2):
            wvec = w_v[pl.ds(g * 16, 16)]
            for j in range(16):
                ws = wvec[j]
                for k2 in range(4):
                    sl = pl.ds(k2 * 16, 16)
                    rows_v[g * 16 + j, sl] = rows_v[g * 16 + j, sl] * ws
            return 0
        lax.fori_loop(0, EB // 16, _row, 0)

        pltpu.sync_copy(rows_v, acc_sh.at[dst_v], add=True)
        return 0

    lax.fori_loop(0, NBAT, _batch, 0)
    plsc.subcore_barrier()
    for j in range(ROWS_PT // EB):
        pltpu.sync_copy(acc_sh.at[pl.ds(s * ROWS_PT + j * EB, EB)], rows_v)
        pltpu.sync_copy(
            rows_v, o2p_hbm.at[pl.ds(c * NP + s * ROWS_PT + j * EB, EB)])


def _scc(src, dst, asad2d, h2cat):
    mesh = plsc.VectorSubcoreMesh(core_axis_name="c", subcore_axis_name="s")
    f = pl.kernel(
        _scc_body,
        out_type=jax.ShapeDtypeStruct((NSC * NP, 128), _f32),
        mesh=mesh,
        compiler_params=pltpu.CompilerParams(needs_layout_passes=False, use_tc_tiling_on_sc=False),
        scratch_types=[
            pltpu.VMEM((NP * 2 // 16, 16), _f32),
            pltpu.VMEM((EB,), _i32),
            pltpu.VMEM((EB,), _i32),
            pltpu.VMEM((EB,), _f32),
            pltpu.VMEM((EB, 128), _f32),
            pltpu.VMEM_SHARED((NP, 128), _f32),
            pltpu.SemaphoreType.DMA,
        ],
    )
    return f(src, dst, asad2d, h2cat)


def _tc3_body(pa_ref, pb_ref, b2_ref, out_ref):
    ps = pa_ref[...] + pb_ref[...]
    lane = lax.broadcasted_iota(_i32, (256, 128), 1)
    d = jnp.sum(jnp.where(lane == 64, ps, 0.0), axis=1, keepdims=True)
    z = ps[:, :64] / (d + 1e-16) + b2_ref[...]
    z = z - jnp.max(z, axis=1, keepdims=True)
    out_ref[...] = z - jnp.log(jnp.sum(jnp.exp(z), axis=1, keepdims=True))


def _tc3(o2p, b2r):
    return pl.pallas_call(
        _tc3_body,
        grid=(NB,),
        in_specs=[
            pl.BlockSpec((256, 128), lambda i: (i, 0)),
            pl.BlockSpec((256, 128), lambda i: (NB + i, 0)),
            pl.BlockSpec((1, 64), lambda i: (0, 0)),
        ],
        out_specs=pl.BlockSpec((256, 64), lambda i: (i, 0)),
        out_shape=jax.ShapeDtypeStruct((NP, NCLS), _f32),
    )(o2p, o2p, b2r)


@jax.jit
def kernel(x, edge_index, W1, a_src1, a_dst1, b1, W2, a_src2, a_dst2, b2):
    xp = jnp.zeros((NP, NFEAT), _f32).at[:N].set(x)

    loop = jnp.arange(N, dtype=_i32)
    padn = jnp.full((EP - E2,), N, dtype=_i32)
    src = jnp.concatenate([edge_index[0].astype(_i32), loop, padn])
    dst = jnp.concatenate([edge_index[1].astype(_i32), loop, padn])

    w1r = W1.reshape(NFEAT, HEADS, HID)
    bsrc = jnp.einsum("khf,hf->kh", w1r, a_src1)
    bdst = jnp.einsum("khf,hf->kh", w1r, a_dst1)
    bbp = jnp.zeros((NFEAT, 128), _f32).at[:, :4].set(bsrc).at[:, 4:8].set(bdst)

    w2cat = jnp.zeros((HEADS * HID, 128), _f32)
    w2cat = w2cat.at[:, :NCLS].set(W2)
    w2cat = w2cat.at[:, 64].set(W2 @ a_src2[0])
    w2cat = w2cat.at[:, 65].set(W2 @ a_dst2[0])

    b1r = b1.reshape(NCH, 1, 128)
    b2r = b2.reshape(1, NCLS)

    h_cm, asad = _tc1(xp, W1, bbp)
    asad2d = asad.reshape(NP * 8 // 16, 16)

    w, denp = _sca(src, dst, asad2d)

    hflat = h_cm.reshape(NCH * NP, CW)
    o1p = _scb(src, dst, w, hflat)

    h2cat, asad2 = _tc2(o1p, denp, b1r, w2cat)
    asad2_2d = asad2.reshape(NP * 2 // 16, 16)

    o2p = _scc(src, dst, asad2_2d, h2cat)

    out = _tc3(o2p, b2r)
    return out[:N]


# speedup vs baseline: 12.5468x; 1.3659x over previous
import functools

import jax
import jax.numpy as jnp
from jax import lax
from jax.experimental import pallas as pl
from jax.experimental.pallas import tpu as pltpu
from jax.experimental.pallas import tpu_sc as plsc

N = 10000
NP = 10240
NB = NP // 256
NFEAT = 256
HEADS = 4
HID = 256
NCH = 8
CW = 128
NCLS = 64
E = 160000
E2 = E + N
EB = 128
EPT = 5376
EP = EPT * 32
NBAT = EPT // EB
NSC = 2
NSUB = 16
ROWS_PT = NP // NSUB

_f32 = jnp.float32
_i32 = jnp.int32


def _tc1_body(x_ref, w1_ref, bb_ref, h_ref, asad_ref):
    xb = x_ref[...]
    h_ref[0] = jnp.dot(xb, w1_ref[...], preferred_element_type=_f32, precision=lax.Precision.HIGHEST)

    @pl.when(pl.program_id(1) == 0)
    def _():
        asad_ref[...] = jnp.dot(xb, bb_ref[...], preferred_element_type=_f32, precision=lax.Precision.HIGHEST)[:, :8]


def _tc1(xp, w1, bbp):
    return pl.pallas_call(
        _tc1_body,
        grid=(NB, NCH),
        in_specs=[
            pl.BlockSpec((256, NFEAT), lambda i, c: (i, 0)),
            pl.BlockSpec((NFEAT, CW), lambda i, c: (0, c)),
            pl.BlockSpec((NFEAT, 128), lambda i, c: (0, 0)),
        ],
        out_specs=[
            pl.BlockSpec((1, 256, CW), lambda i, c: (c, i, 0)),
            pl.BlockSpec((256, 8), lambda i, c: (i, 0)),
        ],
        out_shape=[
            jax.ShapeDtypeStruct((NCH, NP, CW), _f32),
            jax.ShapeDtypeStruct((NP, 8), _f32),
        ],
    )(xp, w1, bbp)


def _sca_body(src_hbm, dst_hbm, asad_hbm, w_hbm, denp_hbm,
              asad_v, src_v, dst_v, wb_v, w16_v, zb_v, dacc_sh, sem):
    c = lax.axis_index("c")
    s = lax.axis_index("s")
    wid = c * NSUB + s

    pltpu.sync_copy(asad_hbm, asad_v)

    def _z1(i, _):
        zb_v[i] = jnp.zeros((16,), _f32)
        return 0
    lax.fori_loop(0, ROWS_PT, _z1, 0)

    def _z2(i, _):
        w16_v[i] = jnp.zeros((16,), _f32)
        return 0
    lax.fori_loop(0, EB, _z2, 0)

    pltpu.sync_copy(zb_v, dacc_sh.at[pl.ds(s * ROWS_PT, ROWS_PT)])
    plsc.subcore_barrier()

    def _mx(i, m):
        return jnp.maximum(m, asad_v[i])
    macc = lax.fori_loop(0, NP * 8 // 16, _mx, jnp.full((16,), -3.4e38, _f32))
    iota16 = lax.iota(_i32, 16)
    gs = []
    for h in range(HEADS):
        vh = jnp.where((iota16 == h) | (iota16 == h + 8), macc, -3.4e38)
        gs.append(plsc.cummax(lax.rev(plsc.cummax(vh), (0,))))

    def _batch(b, _):
        off = wid * EPT + b * EB
        pltpu.sync_copy(src_hbm.at[pl.ds(off, EB)], src_v)
        pltpu.sync_copy(dst_hbm.at[pl.ds(off, EB)], dst_v)
        for h in range(HEADS):
            def _grp(g, _h=h):
                s16 = src_v[pl.ds(g * 16, 16)]
                d16 = dst_v[pl.ds(g * 16, 16)]
                fs = s16 * 8 + _h
                fd = d16 * 8 + (4 + _h)
                asg = plsc.load_gather(asad_v, [fs >> 4, fs & 15])
                adg = plsc.load_gather(asad_v, [fd >> 4, fd & 15])
                al = asg + adg
                al = jnp.where(al > 0, al, 0.2 * al)
                mm = gs[_h] + adg
                mm = jnp.where(mm > 0, mm, 0.2 * mm)
                w = jnp.exp(al - mm)
                wb_v[_h, pl.ds(g * 16, 16)] = w
                plsc.store_scatter(
                    w16_v,
                    [g * 16 + iota16, jnp.full((16,), _h, _i32)],
                    w)

            def _grp_loop(g, _2, _f=_grp):
                _f(g)
                return 0
            lax.fori_loop(0, EB // 16, _grp_loop, 0)
        pltpu.sync_copy(w16_v, dacc_sh.at[dst_v], add=True)
        for h in range(HEADS):
            pltpu.sync_copy(wb_v.at[h], w_hbm.at[h, pl.ds(off, EB)])
        return 0

    lax.fori_loop(0, NBAT, _batch, 0)
    plsc.subcore_barrier()
    for j in range(ROWS_PT // EB):
        pltpu.sync_copy(dacc_sh.at[pl.ds(s * ROWS_PT + j * EB, EB)], w16_v)
        pltpu.sync_copy(
            w16_v, denp_hbm.at[pl.ds(c * NP + s * ROWS_PT + j * EB, EB)])


def _sca(src, dst, asad2d):
    mesh = plsc.VectorSubcoreMesh(core_axis_name="c", subcore_axis_name="s")
    f = pl.kernel(
        _sca_body,
        out_type=[
            jax.ShapeDtypeStruct((HEADS, EP), _f32),
            jax.ShapeDtypeStruct((NSC * NP, 16), _f32),
        ],
        mesh=mesh,
        compiler_params=pltpu.CompilerParams(needs_layout_passes=False, use_tc_tiling_on_sc=False),
        scratch_types=[
            pltpu.VMEM((NP * 8 // 16, 16), _f32),
            pltpu.VMEM((EB,), _i32),
            pltpu.VMEM((EB,), _i32),
            pltpu.VMEM((HEADS, EB), _f32),
            pltpu.VMEM((EB, 16), _f32),
            pltpu.VMEM((ROWS_PT, 16), _f32),
            pltpu.VMEM_SHARED((NP, 16), _f32),
            pltpu.SemaphoreType.DMA,
        ],
    )
    return f(src, dst, asad2d)


def _scb_body(src2_hbm, dst2_hbm, w3_hbm, hflat_hbm, o1p_hbm,
              src2d, dst2d, idx2, w2v, rows0, rows1,
              acc_sh, sg0, sg1, sw0, sw1):
    c = lax.axis_index("c")
    s = lax.axis_index("s")
    rows = (rows0, rows1)
    sg = (sg0, sg1)
    sw = (sw0, sw1)
    roff = c * (EP // 2 // EB) + s * NBAT

    def _chunk_body(chunk, _0):
        hc = chunk // 2
        pltpu.sync_copy(src2_hbm.at[pl.ds(roff, NBAT)], src2d)
        pltpu.sync_copy(dst2_hbm.at[pl.ds(roff, NBAT)], dst2d)

        def _z(i, _):
            for k2 in range(8):
                rows0[i, pl.ds(k2 * 16, 16)] = jnp.zeros((16,), _f32)
            return 0
        lax.fori_loop(0, EB, _z, 0)
        for j in range(ROWS_PT // EB):
            pltpu.sync_copy(rows0, acc_sh.at[pl.ds(s * ROWS_PT + j * EB, EB)])
        plsc.subcore_barrier()

        def _ix(buf, b1):
            for k in range(8):
                sl = pl.ds(k * 16, 16)
                idx2[buf, sl] = src2d[b1, sl] + chunk * NP

        def _prefetch(buf, b1):
            _ix(buf, b1)
            pltpu.async_copy(hflat_hbm.at[idx2.at[buf]], rows[buf], sg[buf])
            pltpu.async_copy(w3_hbm.at[hc, roff + b1], w2v.at[buf], sw[buf])

        def _consume(buf, b):
            pltpu.make_async_copy(
                hflat_hbm.at[idx2.at[buf]], rows[buf], sg[buf]).wait()
            pltpu.make_async_copy(
                w3_hbm.at[hc, roff + b], w2v.at[buf], sw[buf]).wait()

            def _row(g2, _2):
                wvec = w2v[buf, pl.ds(g2 * 16, 16)]
                for jj in range(16):
                    ws = wvec[jj]
                    for k2 in range(8):
                        sl = pl.ds(k2 * 16, 16)
                        rows[buf][g2 * 16 + jj, sl] = (
                            rows[buf][g2 * 16 + jj, sl] * ws)
                return 0
            lax.fori_loop(0, EB // 16, _row, 0)
            pltpu.sync_copy(rows[buf], acc_sh.at[dst2d.at[b]], add=True)

        _prefetch(0, 0)

        def _super(g, _):
            b = g * 2
            _prefetch(1, b + 1)
            _consume(0, b)
            _prefetch(0, b + 2)
            _consume(1, b + 1)
            return 0

        lax.fori_loop(0, NBAT // 2 - 1, _super, 0)
        _prefetch(1, NBAT - 1)
        _consume(0, NBAT - 2)
        _consume(1, NBAT - 1)
        plsc.subcore_barrier()

        base = (c * NCH + chunk) * NP + s * ROWS_PT
        for j in range(ROWS_PT // EB):
            pltpu.sync_copy(acc_sh.at[pl.ds(s * ROWS_PT + j * EB, EB)], rows0)
            pltpu.sync_copy(rows0, o1p_hbm.at[pl.ds(base + j * EB, EB)])
        plsc.subcore_barrier()
        return 0

    lax.fori_loop(0, NCH, _chunk_body, 0)


def _scb(src2, dst2, w3, hflat):
    mesh = plsc.VectorSubcoreMesh(core_axis_name="c", subcore_axis_name="s")
    f = pl.kernel(
        _scb_body,
        out_type=jax.ShapeDtypeStruct((NSC * NCH * NP, CW), _f32),
        mesh=mesh,
        compiler_params=pltpu.CompilerParams(needs_layout_passes=False, use_tc_tiling_on_sc=False),
        scratch_types=[
            pltpu.VMEM((NBAT, EB), _i32),
            pltpu.VMEM((NBAT, EB), _i32),
            pltpu.VMEM((2, EB), _i32),
            pltpu.VMEM((2, EB), _f32),
            pltpu.VMEM((EB, CW), _f32),
            pltpu.VMEM((EB, CW), _f32),
            pltpu.VMEM_SHARED((NP, CW), _f32),
            pltpu.SemaphoreType.DMA,
            pltpu.SemaphoreType.DMA,
            pltpu.SemaphoreType.DMA,
            pltpu.SemaphoreType.DMA,
        ],
    )
    return f(src2, dst2, w3, hflat)


def _tc2_body(o1a_ref, o1b_ref, dpa_ref, dpb_ref, b1_ref, w2_ref,
              h2_ref, asad2_ref):
    k = pl.program_id(1)
    o1 = o1a_ref[...] + o1b_ref[...]
    dall = dpa_ref[...] + dpb_ref[...]
    hk = k // 2
    lane16 = lax.broadcasted_iota(_i32, (256, 16), 1)
    d = jnp.sum(jnp.where(lane16 == hk, dall, 0.0), axis=1, keepdims=True)
    hb = o1 / (d + 1e-16) + b1_ref[0]
    hb = jnp.where(hb > 0, hb, jnp.exp(jnp.minimum(hb, 0.0)) - 1.0)
    mm = jnp.dot(hb, w2_ref[...], preferred_element_type=_f32, precision=lax.Precision.HIGHEST)

    @pl.when(k == 0)
    def _():
        h2_ref[...] = mm

    @pl.when(k > 0)
    def _():
        h2_ref[...] += mm

    @pl.when(k == NCH - 1)
    def _():
        asad2_ref[...] = h2_ref[:, 64:66]


def _tc2(o1p, denp, b1r, w2cat):
    return pl.pallas_call(
        _tc2_body,
        grid=(NB, NCH),
        in_specs=[
            pl.BlockSpec((256, CW), lambda i, k: (k * NB + i, 0)),
            pl.BlockSpec((256, CW), lambda i, k: ((NCH + k) * NB + i, 0)),
            pl.BlockSpec((256, 16), lambda i, k: (i, 0)),
            pl.BlockSpec((256, 16), lambda i, k: (NB + i, 0)),
            pl.BlockSpec((1, 1, 128), lambda i, k: (k, 0, 0)),
            pl.BlockSpec((CW, 128), lambda i, k: (k, 0)),
        ],
        out_specs=[
            pl.BlockSpec((256, 128), lambda i, k: (i, 0)),
            pl.BlockSpec((256, 2), lambda i, k: (i, 0)),
        ],
        out_shape=[
            jax.ShapeDtypeStruct((NP, 128), _f32),
            jax.ShapeDtypeStruct((NP, 2), _f32),
        ],
    )(o1p, o1p, denp, denp, b1r, w2cat)


def _scc_body(src_hbm, dst_hbm, asad_hbm, h2_hbm, o2p_hbm,
              asad_v, src_v, dst_v, w_v, rows_v, acc_sh, sem):
    c = lax.axis_index("c")
    s = lax.axis_index("s")

    pltpu.sync_copy(asad_hbm, asad_v)

    def _z(i, _):
        for k2 in range(8):
            rows_v[i, pl.ds(k2 * 16, 16)] = jnp.zeros((16,), _f32)
        return 0
    lax.fori_loop(0, EB, _z, 0)
    for j in range(ROWS_PT // EB):
        pltpu.sync_copy(rows_v, acc_sh.at[pl.ds(s * ROWS_PT + j * EB, EB)])
    plsc.subcore_barrier()

    def _mx(i, m):
        return jnp.maximum(m, asad_v[i])
    macc = lax.fori_loop(0, NP * 2 // 16, _mx, jnp.full((16,), -3.4e38, _f32))
    iota16 = lax.iota(_i32, 16)
    v2 = jnp.where(iota16 % 2 == 0, macc, -3.4e38)
    g2 = plsc.cummax(lax.rev(plsc.cummax(v2), (0,)))

    def _batch(b, _):
        off = c * (EP // 2) + s * EPT + b * EB
        pltpu.sync_copy(src_hbm.at[pl.ds(off, EB)], src_v)
        pltpu.sync_copy(dst_hbm.at[pl.ds(off, EB)], dst_v)
        pltpu.async_copy(h2_hbm.at[src_v], rows_v, sem).wait()

        def _grp(g, _2):
            s16 = src_v[pl.ds(g * 16, 16)]
            d16 = dst_v[pl.ds(g * 16, 16)]
            fs = s16 * 2
            fd = d16 * 2 + 1
            asg = plsc.load_gather(asad_v, [fs >> 4, fs & 15])
            adg = plsc.load_gather(asad_v, [fd >> 4, fd & 15])
            al = asg + adg
            al = jnp.where(al > 0, al, 0.2 * al)
            mm = g2 + adg
            mm = jnp.where(mm > 0, mm, 0.2 * mm)
            w = jnp.exp(al - mm)
            w_v[pl.ds(g * 16, 16)] = w
            plsc.store_scatter(
                rows_v, [g * 16 + iota16, jnp.full((16,), 64, _i32)], w)
            return 0
        lax.fori_loop(0, EB // 16, _grp, 0)

        def _row(g, _2):
            wvec = w_v[pl.ds(g * 16, 16)]
            for j in range(16):
                ws = wvec[j]
                for k2 in range(4):
                    sl = pl.ds(k2 * 16, 16)
                    rows_v[g * 16 + j, sl] = rows_v[g * 16 + j, sl] * ws
            return 0
        lax.fori_loop(0, EB // 16, _row, 0)

        pltpu.sync_copy(rows_v, acc_sh.at[dst_v], add=True)
        return 0

    lax.fori_loop(0, NBAT, _batch, 0)
    plsc.subcore_barrier()
    for j in range(ROWS_PT // EB):
        pltpu.sync_copy(acc_sh.at[pl.ds(s * ROWS_PT + j * EB, EB)], rows_v)
        pltpu.sync_copy(
            rows_v, o2p_hbm.at[pl.ds(c * NP + s * ROWS_PT + j * EB, EB)])


def _scc(src, dst, asad2d, h2cat):
    mesh = plsc.VectorSubcoreMesh(core_axis_name="c", subcore_axis_name="s")
    f = pl.kernel(
        _scc_body,
        out_type=jax.ShapeDtypeStruct((NSC * NP, 128), _f32),
        mesh=mesh,
        compiler_params=pltpu.CompilerParams(needs_layout_passes=False, use_tc_tiling_on_sc=False),
        scratch_types=[
            pltpu.VMEM((NP * 2 // 16, 16), _f32),
            pltpu.VMEM((EB,), _i32),
            pltpu.VMEM((EB,), _i32),
            pltpu.VMEM((EB,), _f32),
            pltpu.VMEM((EB, 128), _f32),
            pltpu.VMEM_SHARED((NP, 128), _f32),
            pltpu.SemaphoreType.DMA,
        ],
    )
    return f(src, dst, asad2d, h2cat)


def _tc3_body(pa_ref, pb_ref, b2_ref, out_ref):
    ps = pa_ref[...] + pb_ref[...]
    lane = lax.broadcasted_iota(_i32, (256, 128), 1)
    d = jnp.sum(jnp.where(lane == 64, ps, 0.0), axis=1, keepdims=True)
    z = ps[:, :64] / (d + 1e-16) + b2_ref[...]
    z = z - jnp.max(z, axis=1, keepdims=True)
    out_ref[...] = z - jnp.log(jnp.sum(jnp.exp(z), axis=1, keepdims=True))


def _tc3(o2p, b2r):
    return pl.pallas_call(
        _tc3_body,
        grid=(NB,),
        in_specs=[
            pl.BlockSpec((256, 128), lambda i: (i, 0)),
            pl.BlockSpec((256, 128), lambda i: (NB + i, 0)),
            pl.BlockSpec((1, 64), lambda i: (0, 0)),
        ],
        out_specs=pl.BlockSpec((256, 64), lambda i: (i, 0)),
        out_shape=jax.ShapeDtypeStruct((NP, NCLS), _f32),
    )(o2p, o2p, b2r)


@jax.jit
def kernel(x, edge_index, W1, a_src1, a_dst1, b1, W2, a_src2, a_dst2, b2):
    xp = jnp.zeros((NP, NFEAT), _f32).at[:N].set(x)

    loop = jnp.arange(N, dtype=_i32)
    padn = jnp.full((EP - E2,), N, dtype=_i32)
    src = jnp.concatenate([edge_index[0].astype(_i32), loop, padn])
    dst = jnp.concatenate([edge_index[1].astype(_i32), loop, padn])

    w1r = W1.reshape(NFEAT, HEADS, HID)
    bsrc = jnp.einsum("khf,hf->kh", w1r, a_src1)
    bdst = jnp.einsum("khf,hf->kh", w1r, a_dst1)
    bbp = jnp.zeros((NFEAT, 128), _f32).at[:, :4].set(bsrc).at[:, 4:8].set(bdst)

    w2cat = jnp.zeros((HEADS * HID, 128), _f32)
    w2cat = w2cat.at[:, :NCLS].set(W2)
    w2cat = w2cat.at[:, 64].set(W2 @ a_src2[0])
    w2cat = w2cat.at[:, 65].set(W2 @ a_dst2[0])

    b1r = b1.reshape(NCH, 1, 128)
    b2r = b2.reshape(1, NCLS)

    h_cm, asad = _tc1(xp, W1, bbp)
    asad2d = asad.reshape(NP * 8 // 16, 16)

    w, denp = _sca(src, dst, asad2d)

    hflat = h_cm.reshape(NCH * NP, CW)
    o1p = _scb(src.reshape(EP // EB, EB), dst.reshape(EP // EB, EB),
               w.reshape(HEADS, EP // EB, EB), hflat)

    h2cat, asad2 = _tc2(o1p, denp, b1r, w2cat)
    asad2_2d = asad2.reshape(NP * 2 // 16, 16)

    o2p = _scc(src, dst, asad2_2d, h2cat)

    out = _tc3(o2p, b2r)
    return out[:N]
